# initial kernel scaffold (unmeasured)
import jax
import jax.numpy as jnp
from jax import lax
from jax.experimental import pallas as pl
from jax.experimental.pallas import tpu as pltpu

N_DEV = 8

_DeviceIdType = getattr(pl, "DeviceIdType", None) or pltpu.DeviceIdType
MESH = _DeviceIdType.MESH
_sem_signal = getattr(pl, "semaphore_signal", None) or pltpu.semaphore_signal
_sem_wait = getattr(pl, "semaphore_wait", None) or pltpu.semaphore_wait
_CompilerParams = getattr(pltpu, "CompilerParams", None) or pltpu.TPUCompilerParams


def kernel(x, w_mat, scale_x, scale_w):
    m_per, k = x.shape
    _, n_per = w_mat.shape

    x8 = x.astype(jnp.float8_e4m3fn)
    w8 = w_mat.astype(jnp.float8_e5m2)
    scale = (scale_x.reshape(-1)[:1] * scale_w.reshape(-1)[:1]).astype(jnp.float32)

    def body(x_ref, w_ref, s_ref, out_ref, xg_ref, send_sems, recv_sems):
        my = lax.axis_index("i")
        left = lax.rem(my + N_DEV - 1, N_DEV)
        right = lax.rem(my + 1, N_DEV)

        barrier = pltpu.get_barrier_semaphore()
        _sem_signal(barrier, inc=1, device_id=(left,), device_id_type=MESH)
        _sem_signal(barrier, inc=1, device_id=(right,), device_id_type=MESH)
        _sem_wait(barrier, 2)

        xg_ref[pl.ds(my * m_per, m_per), :] = x_ref[...]

        for h in range(N_DEV - 1):
            src = lax.rem(my + N_DEV - h, N_DEV) if h else my
            off = src * m_per
            rdma = pltpu.make_async_remote_copy(
                src_ref=xg_ref.at[pl.ds(off, m_per), :],
                dst_ref=xg_ref.at[pl.ds(off, m_per), :],
                send_sem=send_sems.at[h],
                recv_sem=recv_sems.at[h],
                device_id=(right,),
                device_id_type=MESH,
            )
            rdma.start()
            rdma.wait()

        acc = lax.dot_general(
            xg_ref[...],
            w_ref[...],
            dimension_numbers=(((1,), (0,)), ((), ())),
            preferred_element_type=jnp.float32,
        )
        out_ref[...] = jnp.maximum(acc * s_ref[0], 0.0)

    return pl.pallas_call(
        body,
        out_shape=jax.ShapeDtypeStruct((N_DEV * m_per, n_per), jnp.float32),
        in_specs=[
            pl.BlockSpec(memory_space=pltpu.VMEM),
            pl.BlockSpec(memory_space=pltpu.VMEM),
            pl.BlockSpec(memory_space=pltpu.SMEM),
        ],
        out_specs=pl.BlockSpec(memory_space=pltpu.VMEM),
        scratch_shapes=[
            pltpu.VMEM((N_DEV * m_per, k), jnp.float8_e4m3fn),
            pltpu.SemaphoreType.DMA((N_DEV - 1,)),
            pltpu.SemaphoreType.DMA((N_DEV - 1,)),
        ],
        compiler_params=_CompilerParams(collective_id=0),
    )(x8, w8, scale)


# baseline (device time: 197318 ns/iter reference)
import jax
import jax.numpy as jnp
from jax import lax
from jax.experimental import pallas as pl
from jax.experimental.pallas import tpu as pltpu

N_DEV = 8

_DeviceIdType = getattr(pl, "DeviceIdType", None) or pltpu.DeviceIdType
MESH = _DeviceIdType.MESH
_sem_signal = getattr(pl, "semaphore_signal", None) or pltpu.semaphore_signal
_sem_wait = getattr(pl, "semaphore_wait", None) or pltpu.semaphore_wait
_CompilerParams = getattr(pltpu, "CompilerParams", None) or pltpu.TPUCompilerParams


def kernel(x, w_mat, scale_x, scale_w):
    m_per, k = x.shape
    _, n_per = w_mat.shape

    x8 = x.astype(jnp.float8_e4m3fn)
    w8 = w_mat.astype(jnp.float8_e5m2)
    scale = (scale_x.reshape(-1)[:1] * scale_w.reshape(-1)[:1]).astype(jnp.float32)

    def body(x_ref, w_ref, s_ref, out_ref, xg_ref, send_sems, recv_sems):
        my = lax.axis_index("i")
        left = lax.rem(my + N_DEV - 1, N_DEV)
        right = lax.rem(my + 1, N_DEV)

        barrier = pltpu.get_barrier_semaphore()
        _sem_signal(barrier, inc=1, device_id=(left,), device_id_type=MESH)
        _sem_signal(barrier, inc=1, device_id=(right,), device_id_type=MESH)
        _sem_wait(barrier, 2)

        xg_ref[pl.ds(my * m_per, m_per), :] = x_ref[...]

        for h in range(N_DEV - 1):
            src = lax.rem(my + N_DEV - h, N_DEV) if h else my
            off = src * m_per
            rdma = pltpu.make_async_remote_copy(
                src_ref=xg_ref.at[pl.ds(off, m_per), :],
                dst_ref=xg_ref.at[pl.ds(off, m_per), :],
                send_sem=send_sems.at[h],
                recv_sem=recv_sems.at[h],
                device_id=(right,),
                device_id_type=MESH,
            )
            rdma.start()
            rdma.wait()

        acc = lax.dot_general(
            xg_ref[...],
            w_ref[...],
            dimension_numbers=(((1,), (0,)), ((), ())),
            preferred_element_type=jnp.float32,
        )
        out_ref[...] = jnp.maximum(acc * s_ref[0], 0.0)

    return pl.pallas_call(
        body,
        out_shape=jax.ShapeDtypeStruct((N_DEV * m_per, n_per), jnp.float32),
        in_specs=[
            pl.BlockSpec(memory_space=pltpu.VMEM),
            pl.BlockSpec(memory_space=pltpu.VMEM),
            pl.BlockSpec(memory_space=pltpu.SMEM),
        ],
        out_specs=pl.BlockSpec(memory_space=pltpu.VMEM),
        scratch_shapes=[
            pltpu.VMEM((N_DEV * m_per, k), jnp.float8_e4m3fn),
            pltpu.SemaphoreType.DMA((N_DEV - 1,)),
            pltpu.SemaphoreType.DMA((N_DEV - 1,)),
        ],
        compiler_params=_CompilerParams(
            collective_id=0, vmem_limit_bytes=60 * 1024 * 1024
        ),
    )(x8, w8, scale)


# device time: 116420 ns/iter; 1.6949x vs baseline; 1.6949x over previous
import jax
import jax.numpy as jnp
from jax import lax
from jax.experimental import pallas as pl
from jax.experimental.pallas import tpu as pltpu

N_DEV = 8

_DeviceIdType = getattr(pl, "DeviceIdType", None) or pltpu.DeviceIdType
MESH = _DeviceIdType.MESH
_sem_signal = getattr(pl, "semaphore_signal", None) or pltpu.semaphore_signal
_sem_wait = getattr(pl, "semaphore_wait", None) or pltpu.semaphore_wait
_CompilerParams = getattr(pltpu, "CompilerParams", None) or pltpu.TPUCompilerParams


def kernel(x, w_mat, scale_x, scale_w):
    m_per, k = x.shape
    _, n_per = w_mat.shape
    half = m_per // 2

    x8 = x.astype(jnp.float8_e4m3fn)
    w8 = w_mat.astype(jnp.float8_e5m2)
    scale = (scale_x.reshape(-1)[:1] * scale_w.reshape(-1)[:1]).astype(jnp.float32)

    def body(x_ref, w_ref, s_ref, out_ref, xg_ref,
             cw_send, cw_recv, ccw_send, ccw_recv):
        my = lax.axis_index("i")
        left = lax.rem(my + N_DEV - 1, N_DEV)
        right = lax.rem(my + 1, N_DEV)

        barrier = pltpu.get_barrier_semaphore()
        _sem_signal(barrier, inc=1, device_id=(left,), device_id_type=MESH)
        _sem_signal(barrier, inc=1, device_id=(right,), device_id_type=MESH)
        _sem_wait(barrier, 2)

        xg_ref[pl.ds(my * m_per, m_per), :] = x_ref[...]

        def gemm(row0, nrows):
            acc = lax.dot_general(
                xg_ref[pl.ds(row0, nrows), :],
                w_ref[...],
                dimension_numbers=(((1,), (0,)), ((), ())),
                preferred_element_type=jnp.float32,
            )
            out_ref[pl.ds(row0, nrows), :] = jnp.maximum(acc * s_ref[0], 0.0)

        for h in range(N_DEV - 1):
            cw_o = lax.rem(my + N_DEV - h, N_DEV)
            ccw_o = lax.rem(my + h, N_DEV)
            cw_off = cw_o * m_per
            ccw_off = ccw_o * m_per + half
            rdma_cw = pltpu.make_async_remote_copy(
                src_ref=xg_ref.at[pl.ds(cw_off, half), :],
                dst_ref=xg_ref.at[pl.ds(cw_off, half), :],
                send_sem=cw_send.at[h],
                recv_sem=cw_recv.at[h],
                device_id=(right,),
                device_id_type=MESH,
            )
            rdma_ccw = pltpu.make_async_remote_copy(
                src_ref=xg_ref.at[pl.ds(ccw_off, half), :],
                dst_ref=xg_ref.at[pl.ds(ccw_off, half), :],
                send_sem=ccw_send.at[h],
                recv_sem=ccw_recv.at[h],
                device_id=(left,),
                device_id_type=MESH,
            )
            rdma_cw.start()
            rdma_ccw.start()

            if h == 0:
                gemm(my * m_per, m_per)
            else:
                gemm(cw_off, half)
                gemm(ccw_off, half)

            rdma_cw.wait()
            rdma_ccw.wait()

        gemm(lax.rem(my + 1, N_DEV) * m_per, half)
        gemm(lax.rem(my + N_DEV - 1, N_DEV) * m_per + half, half)

    return pl.pallas_call(
        body,
        out_shape=jax.ShapeDtypeStruct((N_DEV * m_per, n_per), jnp.float32),
        in_specs=[
            pl.BlockSpec(memory_space=pltpu.VMEM),
            pl.BlockSpec(memory_space=pltpu.VMEM),
            pl.BlockSpec(memory_space=pltpu.SMEM),
        ],
        out_specs=pl.BlockSpec(memory_space=pltpu.VMEM),
        scratch_shapes=[
            pltpu.VMEM((N_DEV * m_per, k), jnp.float8_e4m3fn),
            pltpu.SemaphoreType.DMA((N_DEV - 1,)),
            pltpu.SemaphoreType.DMA((N_DEV - 1,)),
            pltpu.SemaphoreType.DMA((N_DEV - 1,)),
            pltpu.SemaphoreType.DMA((N_DEV - 1,)),
        ],
        compiler_params=_CompilerParams(
            collective_id=0, vmem_limit_bytes=60 * 1024 * 1024
        ),
    )(x8, w8, scale)


# device time: 103878 ns/iter; 1.8995x vs baseline; 1.1207x over previous
import jax
import jax.numpy as jnp
from jax import lax
from jax.experimental import pallas as pl
from jax.experimental.pallas import tpu as pltpu

N_DEV = 8
N_PIECE = 2

_DeviceIdType = getattr(pl, "DeviceIdType", None) or pltpu.DeviceIdType
MESH = _DeviceIdType.MESH
_sem_signal = getattr(pl, "semaphore_signal", None) or pltpu.semaphore_signal
_sem_wait = getattr(pl, "semaphore_wait", None) or pltpu.semaphore_wait
_CompilerParams = getattr(pltpu, "CompilerParams", None) or pltpu.TPUCompilerParams


def kernel(x, w_mat, scale_x, scale_w):
    m_per, k = x.shape
    _, n_per = w_mat.shape
    half = m_per // 2
    piece = half // N_PIECE

    x8 = x.astype(jnp.float8_e4m3fn)
    w8 = w_mat.astype(jnp.float8_e5m2)
    scale = (scale_x.reshape(-1)[:1] * scale_w.reshape(-1)[:1]).astype(jnp.float32)

    def body(x_ref, w_ref, s_ref, out_ref, xg_ref,
             cw_send, cw_recv, ccw_send, ccw_recv):
        my = lax.axis_index("i")
        left = lax.rem(my + N_DEV - 1, N_DEV)
        right = lax.rem(my + 1, N_DEV)

        barrier = pltpu.get_barrier_semaphore()
        _sem_signal(barrier, inc=1, device_id=(left,), device_id_type=MESH)
        _sem_signal(barrier, inc=1, device_id=(right,), device_id_type=MESH)
        _sem_wait(barrier, 2)

        xg_ref[pl.ds(my * m_per, m_per), :] = x_ref[...]

        def cw_piece(h, p):
            o = lax.rem(my + N_DEV - h, N_DEV)
            return o * m_per + p * piece

        def ccw_piece(h, p):
            o = lax.rem(my + h, N_DEV)
            return o * m_per + half + p * piece

        def rdma(off, send_sem, recv_sem, dev):
            return pltpu.make_async_remote_copy(
                src_ref=xg_ref.at[pl.ds(off, piece), :],
                dst_ref=xg_ref.at[pl.ds(off, piece), :],
                send_sem=send_sem,
                recv_sem=recv_sem,
                device_id=(dev,),
                device_id_type=MESH,
            )

        def cw(h, p):
            return rdma(cw_piece(h, p), cw_send.at[h, p], cw_recv.at[h, p], right)

        def ccw(h, p):
            return rdma(ccw_piece(h, p), ccw_send.at[h, p], ccw_recv.at[h, p], left)

        def gemm(row0, nrows):
            acc = lax.dot_general(
                xg_ref[pl.ds(row0, nrows), :],
                w_ref[...],
                dimension_numbers=(((1,), (0,)), ((), ())),
                preferred_element_type=jnp.float32,
            )
            out_ref[pl.ds(row0, nrows), :] = jnp.maximum(acc * s_ref[0], 0.0)

        for p in range(N_PIECE):
            cw(0, p).start()
            ccw(0, p).start()
        gemm(my * m_per, m_per)

        for h in range(1, N_DEV - 1):
            for p in range(N_PIECE):
                cw(h - 1, p).wait_recv()
                cw(h, p).start()
                ccw(h - 1, p).wait_recv()
                ccw(h, p).start()
            for p in range(N_PIECE):
                cw(h - 1, p).wait_send()
                ccw(h - 1, p).wait_send()
            gemm(lax.rem(my + N_DEV - h, N_DEV) * m_per, half)
            gemm(lax.rem(my + h, N_DEV) * m_per + half, half)

        last = N_DEV - 2
        for p in range(N_PIECE):
            cw(last, p).wait_recv()
            ccw(last, p).wait_recv()
            cw(last, p).wait_send()
            ccw(last, p).wait_send()
        gemm(lax.rem(my + 1, N_DEV) * m_per, half)
        gemm(lax.rem(my + N_DEV - 1, N_DEV) * m_per + half, half)

    return pl.pallas_call(
        body,
        out_shape=jax.ShapeDtypeStruct((N_DEV * m_per, n_per), jnp.float32),
        in_specs=[
            pl.BlockSpec(memory_space=pltpu.VMEM),
            pl.BlockSpec(memory_space=pltpu.VMEM),
            pl.BlockSpec(memory_space=pltpu.SMEM),
        ],
        out_specs=pl.BlockSpec(memory_space=pltpu.VMEM),
        scratch_shapes=[
            pltpu.VMEM((N_DEV * m_per, k), jnp.float8_e4m3fn),
            pltpu.SemaphoreType.DMA((N_DEV - 1, N_PIECE)),
            pltpu.SemaphoreType.DMA((N_DEV - 1, N_PIECE)),
            pltpu.SemaphoreType.DMA((N_DEV - 1, N_PIECE)),
            pltpu.SemaphoreType.DMA((N_DEV - 1, N_PIECE)),
        ],
        compiler_params=_CompilerParams(
            collective_id=0, vmem_limit_bytes=60 * 1024 * 1024
        ),
    )(x8, w8, scale)


# device time: 103859 ns/iter; 1.8999x vs baseline; 1.0002x over previous
import jax
import jax.numpy as jnp
from jax import lax
from jax.experimental import pallas as pl
from jax.experimental.pallas import tpu as pltpu

N_DEV = 8
N_PIECE = 2

_DeviceIdType = getattr(pl, "DeviceIdType", None) or pltpu.DeviceIdType
MESH = _DeviceIdType.MESH
_sem_signal = getattr(pl, "semaphore_signal", None) or pltpu.semaphore_signal
_sem_wait = getattr(pl, "semaphore_wait", None) or pltpu.semaphore_wait
_CompilerParams = getattr(pltpu, "CompilerParams", None) or pltpu.TPUCompilerParams


def kernel(x, w_mat, scale_x, scale_w):
    m_per, k = x.shape
    _, n_per = w_mat.shape
    half = m_per // 2
    piece = half // N_PIECE

    x8 = x.astype(jnp.float8_e4m3fn)
    w8 = w_mat.astype(jnp.float8_e5m2)
    scale = (scale_x.reshape(-1)[:1] * scale_w.reshape(-1)[:1]).astype(jnp.float32)

    def body(x_ref, w_ref, s_ref, out_ref, xg_ref,
             cw_send, cw_recv, ccw_send, ccw_recv):
        my = lax.axis_index("i")
        left = lax.rem(my + N_DEV - 1, N_DEV)
        right = lax.rem(my + 1, N_DEV)

        barrier = pltpu.get_barrier_semaphore()
        _sem_signal(barrier, inc=1, device_id=(left,), device_id_type=MESH)
        _sem_signal(barrier, inc=1, device_id=(right,), device_id_type=MESH)
        _sem_wait(barrier, 2)

        xg_ref[pl.ds(my * m_per, m_per), :] = x_ref[...]

        def cw_piece(h, p):
            o = lax.rem(my + N_DEV - h, N_DEV)
            return o * m_per + p * piece

        def ccw_piece(h, p):
            o = lax.rem(my + h, N_DEV)
            return o * m_per + half + p * piece

        def rdma(off, send_sem, recv_sem, dev):
            return pltpu.make_async_remote_copy(
                src_ref=xg_ref.at[pl.ds(off, piece), :],
                dst_ref=xg_ref.at[pl.ds(off, piece), :],
                send_sem=send_sem,
                recv_sem=recv_sem,
                device_id=(dev,),
                device_id_type=MESH,
            )

        def cw(h, p):
            return rdma(cw_piece(h, p), cw_send.at[h, p], cw_recv.at[h, p], right)

        def ccw(h, p):
            return rdma(ccw_piece(h, p), ccw_send.at[h, p], ccw_recv.at[h, p], left)

        def gemm(row0, nrows):
            acc = lax.dot_general(
                xg_ref[pl.ds(row0, nrows), :],
                w_ref[...],
                dimension_numbers=(((1,), (0,)), ((), ())),
                preferred_element_type=jnp.float32,
            )
            out_ref[pl.ds(row0, nrows), :] = jnp.maximum(acc * s_ref[0], 0.0)

        for p in range(N_PIECE):
            cw(0, p).start()
            ccw(0, p).start()

        for h in range(1, N_DEV - 1):
            cw(h - 1, 0).wait_recv()
            cw(h, 0).start()
            ccw(h - 1, 0).wait_recv()
            ccw(h, 0).start()
            gemm(lax.rem(my + N_DEV - (h - 1), N_DEV) * m_per, half)
            cw(h - 1, 1).wait_recv()
            cw(h, 1).start()
            ccw(h - 1, 1).wait_recv()
            ccw(h, 1).start()
            gemm(lax.rem(my + h - 1, N_DEV) * m_per + half, half)
            for p in range(N_PIECE):
                cw(h - 1, p).wait_send()
                ccw(h - 1, p).wait_send()

        last = N_DEV - 2
        gemm(lax.rem(my + N_DEV - last, N_DEV) * m_per, half)
        gemm(lax.rem(my + last, N_DEV) * m_per + half, half)
        for p in range(N_PIECE):
            cw(last, p).wait_recv()
            ccw(last, p).wait_recv()
            cw(last, p).wait_send()
            ccw(last, p).wait_send()
        gemm(lax.rem(my + 1, N_DEV) * m_per, half)
        gemm(lax.rem(my + N_DEV - 1, N_DEV) * m_per + half, half)

    return pl.pallas_call(
        body,
        out_shape=jax.ShapeDtypeStruct((N_DEV * m_per, n_per), jnp.float32),
        in_specs=[
            pl.BlockSpec(memory_space=pltpu.VMEM),
            pl.BlockSpec(memory_space=pltpu.VMEM),
            pl.BlockSpec(memory_space=pltpu.SMEM),
        ],
        out_specs=pl.BlockSpec(memory_space=pltpu.VMEM),
        scratch_shapes=[
            pltpu.VMEM((N_DEV * m_per, k), jnp.float8_e4m3fn),
            pltpu.SemaphoreType.DMA((N_DEV - 1, N_PIECE)),
            pltpu.SemaphoreType.DMA((N_DEV - 1, N_PIECE)),
            pltpu.SemaphoreType.DMA((N_DEV - 1, N_PIECE)),
            pltpu.SemaphoreType.DMA((N_DEV - 1, N_PIECE)),
        ],
        compiler_params=_CompilerParams(
            collective_id=0, vmem_limit_bytes=60 * 1024 * 1024
        ),
    )(x8, w8, scale)
